# Initial kernel scaffold; baseline (speedup 1.0000x reference)
import functools

import jax
import jax.numpy as jnp
from jax import lax
from jax.experimental import pallas as pl
from jax.experimental.pallas import tpu as pltpu
from jax.experimental.pallas import tpu_sc as plsc

N_NODES = 10000
N_EDGES = 320000
D = 128
HID = 128

NC = 2
NS = 16
NW = NC * NS
EPT = N_EDGES // NW
CH = 80
NCH = EPT // CH
RPT = N_NODES // NS

_mesh = plsc.VectorSubcoreMesh(core_axis_name="c", subcore_axis_name="s")


@functools.partial(
    pl.kernel,
    out_type=(
        jax.ShapeDtypeStruct((NC * N_NODES, D), jnp.float32),
        jax.ShapeDtypeStruct((NC * N_NODES, 16), jnp.float32),
    ),
    mesh=_mesh,
    scratch_types=[
        pltpu.VMEM((EPT,), jnp.int32),
        pltpu.VMEM((NCH, CH), jnp.int32),
        pltpu.VMEM((CH, D), jnp.float32),
        pltpu.VMEM((CH, 16), jnp.float32),
        pltpu.VMEM_SHARED((N_NODES, D), jnp.float32),
        pltpu.VMEM_SHARED((N_NODES, 16), jnp.float32),
        pltpu.SemaphoreType.DMA,
    ],
)
def _sc_aggregate(src_hbm, tgt3_hbm, nf_hbm, zrows_hbm, zcnt_hbm, ones_hbm,
                  psum_hbm, pcnt_hbm,
                  src_v, tgt_v, rows_v, ones_v, acc_sum, acc_cnt, sem):
    c = lax.axis_index("c")
    s = lax.axis_index("s")
    wid = c * NS + s
    ebase = wid * EPT
    rbase = s * RPT

    pltpu.sync_copy(zrows_hbm, acc_sum.at[pl.ds(rbase, RPT)])
    pltpu.sync_copy(zcnt_hbm, acc_cnt.at[pl.ds(rbase, RPT)])
    pltpu.sync_copy(src_hbm.at[pl.ds(ebase, EPT)], src_v)
    pltpu.sync_copy(tgt3_hbm.at[wid], tgt_v)
    pltpu.sync_copy(ones_hbm, ones_v)
    plsc.subcore_barrier()

    def chunk(j, carry):
        off = pl.multiple_of(j * CH, 8)
        pltpu.async_copy(nf_hbm.at[src_v.at[pl.ds(off, CH)]], rows_v, sem).wait()
        pltpu.sync_copy(rows_v, acc_sum.at[tgt_v.at[j]], add=True)
        pltpu.sync_copy(ones_v, acc_cnt.at[tgt_v.at[j]], add=True)
        return carry

    lax.fori_loop(0, NCH, chunk, 0)
    plsc.subcore_barrier()

    obase = c * N_NODES + rbase
    pltpu.sync_copy(acc_sum.at[pl.ds(rbase, RPT)], psum_hbm.at[pl.ds(obase, RPT)])
    pltpu.sync_copy(acc_cnt.at[pl.ds(rbase, RPT)], pcnt_hbm.at[pl.ds(obase, RPT)])


def _tc_mlp_body(p0, p1, c0, c1, nf, w1a, w1b, b1, w2, b2, out):
    cnt = jnp.maximum(c0[:, 0:1] + c1[:, 0:1], 1.0)
    agg = (p0[...] + p1[...]) / cnt
    h = (
        lax.dot_general(agg, w1a[...], (((1,), (1,)), ((), ())),
                        preferred_element_type=jnp.float32)
        + lax.dot_general(nf[...], w1b[...], (((1,), (1,)), ((), ())),
                          preferred_element_type=jnp.float32)
        + b1[...]
    )
    h = jnp.maximum(h, 0.0)
    out[...] = (
        lax.dot_general(h, w2[...], (((1,), (1,)), ((), ())),
                        preferred_element_type=jnp.float32)
        + b2[...]
    )


def kernel(node_features, edge_index, W1, b1, W2, b2):
    src = edge_index[0].astype(jnp.int32)
    tgt3 = edge_index[1].astype(jnp.int32).reshape(NW, NCH, CH)
    zrows = jnp.zeros((RPT, D), jnp.float32)
    zcnt = jnp.zeros((RPT, 16), jnp.float32)
    ones = jnp.ones((CH, 16), jnp.float32)

    psum, pcnt = _sc_aggregate(src, tgt3, node_features, zrows, zcnt, ones)

    R = 1000
    grid = (N_NODES // R,)
    out = pl.pallas_call(
        _tc_mlp_body,
        grid=grid,
        in_specs=[
            pl.BlockSpec((R, D), lambda i: (i, 0)),
            pl.BlockSpec((R, D), lambda i: (i, 0)),
            pl.BlockSpec((R, 16), lambda i: (i, 0)),
            pl.BlockSpec((R, 16), lambda i: (i, 0)),
            pl.BlockSpec((R, D), lambda i: (i, 0)),
            pl.BlockSpec((HID, D), lambda i: (0, 0)),
            pl.BlockSpec((HID, D), lambda i: (0, 0)),
            pl.BlockSpec((1, HID), lambda i: (0, 0)),
            pl.BlockSpec((D, HID), lambda i: (0, 0)),
            pl.BlockSpec((1, D), lambda i: (0, 0)),
        ],
        out_specs=pl.BlockSpec((R, D), lambda i: (i, 0)),
        out_shape=jax.ShapeDtypeStruct((N_NODES, D), jnp.float32),
    )(
        psum[:N_NODES], psum[N_NODES:],
        pcnt[:N_NODES], pcnt[N_NODES:],
        node_features,
        W1[:, :D], W1[:, D:],
        b1.reshape(1, HID), W2, b2.reshape(1, D),
    )
    return out


# speedup vs baseline: 3.5649x; 3.5649x over previous
import functools

import jax
import jax.numpy as jnp
from jax import lax
from jax.experimental import pallas as pl
from jax.experimental.pallas import tpu as pltpu
from jax.experimental.pallas import tpu_sc as plsc

N_NODES = 10000
N_EDGES = 320000
D = 128
HID = 128

NC = 2
NS = 16
NW = NC * NS
EPT = N_EDGES // NW
CH = 128
NCH = 80
EPT_P = NCH * CH
N_PAD = 10240
RPT = N_PAD // NS
WCH = RPT // CH

_mesh = plsc.VectorSubcoreMesh(core_axis_name="c", subcore_axis_name="s")


@functools.partial(
    pl.kernel,
    out_type=(
        jax.ShapeDtypeStruct((NC * N_PAD, D), jnp.float32),
        jax.ShapeDtypeStruct((NC * N_PAD, D), jnp.float32),
    ),
    mesh=_mesh,
    scratch_types=[
        pltpu.VMEM((EPT_P,), jnp.int32),
        pltpu.VMEM((NCH, CH), jnp.int32),
        pltpu.VMEM((CH, D), jnp.float32),
        pltpu.VMEM_SHARED((N_PAD, D), jnp.float32),
        pltpu.SemaphoreType.DMA,
    ],
)
def _sc_aggregate(src_hbm, tgt2_hbm, nf_hbm, zrows_hbm, ones_hbm,
                  psum_hbm, pcnt_hbm,
                  src_v, tgt_v, rows_v, acc, sem):
    c = lax.axis_index("c")
    s = lax.axis_index("s")
    wid = c * NS + s
    rbase = s * RPT
    obase = c * N_PAD + rbase

    def zero_acc(_):
        pltpu.sync_copy(zrows_hbm, rows_v)

        def zero_chunk(k, carry):
            roff = pl.multiple_of(rbase + k * CH, 8)
            pltpu.sync_copy(rows_v, acc.at[pl.ds(roff, CH)])
            return carry

        lax.fori_loop(0, WCH, zero_chunk, 0)

    def write_acc(out_hbm):
        def write_chunk(k, carry):
            roff = pl.multiple_of(rbase + k * CH, 8)
            ooff = pl.multiple_of(obase + k * CH, 8)
            pltpu.sync_copy(acc.at[pl.ds(roff, CH)], rows_v)
            pltpu.sync_copy(rows_v, out_hbm.at[pl.ds(ooff, CH)])
            return carry

        lax.fori_loop(0, WCH, write_chunk, 0)

    pltpu.sync_copy(src_hbm.at[pl.ds(wid * EPT_P, EPT_P)], src_v)
    pltpu.sync_copy(tgt2_hbm.at[pl.ds(wid * NCH, NCH)], tgt_v)

    zero_acc(None)
    plsc.subcore_barrier()

    def sum_chunk(j, carry):
        off = pl.multiple_of(j * CH, 8)
        pltpu.async_copy(nf_hbm.at[src_v.at[pl.ds(off, CH)]], rows_v,
                         sem).wait()
        pltpu.sync_copy(rows_v, acc.at[tgt_v.at[j]], add=True)
        return carry

    lax.fori_loop(0, NCH, sum_chunk, 0)
    plsc.subcore_barrier()
    write_acc(psum_hbm)
    plsc.subcore_barrier()

    zero_acc(None)
    plsc.subcore_barrier()
    pltpu.sync_copy(ones_hbm, rows_v)

    def cnt_chunk(j, carry):
        pltpu.sync_copy(rows_v, acc.at[tgt_v.at[j]], add=True)
        return carry

    lax.fori_loop(0, NCH, cnt_chunk, 0)
    plsc.subcore_barrier()
    write_acc(pcnt_hbm)


def _tc_mlp_body(p0, p1, c0, c1, nf, w1a, w1b, b1, w2, b2, out):
    inv = 1.0 / jnp.maximum(c0[:, 0:1] + c1[:, 0:1], 1.0)
    agg = (p0[...] + p1[...]) * inv
    h = (
        lax.dot_general(agg, w1a[...], (((1,), (1,)), ((), ())),
                        preferred_element_type=jnp.float32)
        + lax.dot_general(nf[...], w1b[...], (((1,), (1,)), ((), ())),
                          preferred_element_type=jnp.float32)
        + b1[...]
    )
    h = jnp.maximum(h, 0.0)
    out[...] = (
        lax.dot_general(h, w2[...], (((1,), (1,)), ((), ())),
                        preferred_element_type=jnp.float32)
        + b2[...]
    )


def kernel(node_features, edge_index, W1, b1, W2, b2):
    src = edge_index[0].astype(jnp.int32).reshape(NW, EPT)
    src = jnp.pad(src, ((0, 0), (0, EPT_P - EPT))).reshape(-1)
    tgt = edge_index[1].astype(jnp.int32).reshape(NW, EPT)
    tgt = jnp.pad(tgt, ((0, 0), (0, EPT_P - EPT)),
                  constant_values=N_PAD - 1).reshape(NW * NCH, CH)
    zrows = jnp.zeros((CH, D), jnp.float32)
    ones = jnp.ones((CH, D), jnp.float32)

    psum, pcnt = _sc_aggregate(src, tgt, node_features, zrows, ones)

    R = 1000
    grid = (N_NODES // R,)
    out = pl.pallas_call(
        _tc_mlp_body,
        grid=grid,
        in_specs=[
            pl.BlockSpec((R, D), lambda i: (i, 0)),
            pl.BlockSpec((R, D), lambda i: (i, 0)),
            pl.BlockSpec((R, D), lambda i: (i, 0)),
            pl.BlockSpec((R, D), lambda i: (i, 0)),
            pl.BlockSpec((R, D), lambda i: (i, 0)),
            pl.BlockSpec((HID, D), lambda i: (0, 0)),
            pl.BlockSpec((HID, D), lambda i: (0, 0)),
            pl.BlockSpec((1, HID), lambda i: (0, 0)),
            pl.BlockSpec((D, HID), lambda i: (0, 0)),
            pl.BlockSpec((1, D), lambda i: (0, 0)),
        ],
        out_specs=pl.BlockSpec((R, D), lambda i: (i, 0)),
        out_shape=jax.ShapeDtypeStruct((N_NODES, D), jnp.float32),
    )(
        psum[:N_NODES], psum[N_PAD:N_PAD + N_NODES],
        pcnt[:N_NODES], pcnt[N_PAD:N_PAD + N_NODES],
        node_features,
        W1[:, :D], W1[:, D:],
        b1.reshape(1, HID), W2, b2.reshape(1, D),
    )
    return out


# speedup vs baseline: 3.9488x; 1.1077x over previous
import functools

import jax
import jax.numpy as jnp
from jax import lax
from jax.experimental import pallas as pl
from jax.experimental.pallas import tpu as pltpu
from jax.experimental.pallas import tpu_sc as plsc

N_NODES = 10000
N_EDGES = 320000
D = 128
HID = 128

NC = 2
NS = 16
NW = NC * NS
EPT = N_EDGES // NW
CH = 128
NCH = 80
NPAIR = NCH // 2
EPT_P = NCH * CH
N_PAD = 10240
RPT = N_PAD // NS
WCH = RPT // CH

_mesh = plsc.VectorSubcoreMesh(core_axis_name="c", subcore_axis_name="s")


@functools.partial(
    pl.kernel,
    out_type=(
        jax.ShapeDtypeStruct((NC * N_PAD, D), jnp.float32),
        jax.ShapeDtypeStruct((NC * N_PAD, D), jnp.float32),
    ),
    mesh=_mesh,
    scratch_types=[
        pltpu.VMEM((EPT_P,), jnp.int32),
        pltpu.VMEM((1, CH), jnp.int32),
        pltpu.VMEM((1, CH), jnp.int32),
        pltpu.VMEM((CH, D), jnp.float32),
        pltpu.VMEM((CH, D), jnp.float32),
        pltpu.VMEM_SHARED((N_PAD, D), jnp.float32),
        pltpu.SemaphoreType.DMA,
        pltpu.SemaphoreType.DMA,
    ],
)
def _sc_aggregate(src_hbm, tgt3_hbm, nf_hbm, zrows_hbm, ones_hbm,
                  psum_hbm, pcnt_hbm,
                  src_v, tgt_c0, tgt_c1, rows0, rows1, acc, sem0, sem1):
    c = lax.axis_index("c")
    s = lax.axis_index("s")
    wid = c * NS + s
    rbase = s * RPT
    obase = c * N_PAD + rbase
    tbase = wid * NCH

    def gather(j, buf, sem):
        off = pl.multiple_of(j * CH, 8)
        return pltpu.async_copy(nf_hbm.at[src_v.at[pl.ds(off, CH)]], buf, sem)

    def zero_acc():
        pltpu.sync_copy(zrows_hbm, rows0)

        def zero_chunk(k, carry):
            roff = pl.multiple_of(rbase + k * CH, 8)
            pltpu.sync_copy(rows0, acc.at[pl.ds(roff, CH)])
            return carry

        lax.fori_loop(0, WCH, zero_chunk, 0)

    def write_acc(out_hbm):
        def write_chunk(k, carry):
            roff = pl.multiple_of(rbase + k * CH, 8)
            ooff = pl.multiple_of(obase + k * CH, 8)
            pltpu.sync_copy(acc.at[pl.ds(roff, CH)], rows0)
            pltpu.sync_copy(rows0, out_hbm.at[pl.ds(ooff, CH)])
            return carry

        lax.fori_loop(0, WCH, write_chunk, 0)

    pltpu.sync_copy(src_hbm.at[pl.ds(wid * EPT_P, EPT_P)], src_v)
    zero_acc()
    plsc.subcore_barrier()

    gather(0, rows0, sem0)

    def sum_pair(jj, carry):
        j0 = jj * 2
        j1 = j0 + 1
        gather(j1, rows1, sem1)
        pltpu.sync_copy(tgt3_hbm.at[tbase + j0], tgt_c0)
        pltpu.make_async_copy(nf_hbm.at[src_v.at[pl.ds(0, CH)]], rows0,
                              sem0).wait()
        pltpu.sync_copy(rows0, acc.at[tgt_c0.at[0]], add=True)

        @pl.when(jj < NPAIR - 1)
        def _():
            gather(j0 + 2, rows0, sem0)

        pltpu.sync_copy(tgt3_hbm.at[tbase + j1], tgt_c1)
        pltpu.make_async_copy(nf_hbm.at[src_v.at[pl.ds(0, CH)]], rows1,
                              sem1).wait()
        pltpu.sync_copy(rows1, acc.at[tgt_c1.at[0]], add=True)
        return carry

    lax.fori_loop(0, NPAIR, sum_pair, 0)
    plsc.subcore_barrier()
    write_acc(psum_hbm)
    plsc.subcore_barrier()

    zero_acc()
    plsc.subcore_barrier()
    pltpu.sync_copy(ones_hbm, rows0)
    pltpu.async_copy(tgt3_hbm.at[tbase], tgt_c0, sem0)

    def cnt_pair(jj, carry):
        j0 = jj * 2
        j1 = j0 + 1
        pltpu.async_copy(tgt3_hbm.at[tbase + j1], tgt_c1, sem1)
        pltpu.make_async_copy(tgt3_hbm.at[tbase], tgt_c0, sem0).wait()
        pltpu.sync_copy(rows0, acc.at[tgt_c0.at[0]], add=True)

        @pl.when(jj < NPAIR - 1)
        def _():
            pltpu.async_copy(tgt3_hbm.at[tbase + j0 + 2], tgt_c0, sem0)

        pltpu.make_async_copy(tgt3_hbm.at[tbase], tgt_c1, sem1).wait()
        pltpu.sync_copy(rows0, acc.at[tgt_c1.at[0]], add=True)
        return carry

    lax.fori_loop(0, NPAIR, cnt_pair, 0)
    plsc.subcore_barrier()
    write_acc(pcnt_hbm)


def _tc_mlp_body(p0, p1, c0, c1, nf, w1a, w1b, b1, w2, b2, out):
    inv = 1.0 / jnp.maximum(c0[:, 0:1] + c1[:, 0:1], 1.0)
    agg = (p0[...] + p1[...]) * inv
    h = (
        lax.dot_general(agg, w1a[...], (((1,), (1,)), ((), ())),
                        preferred_element_type=jnp.float32)
        + lax.dot_general(nf[...], w1b[...], (((1,), (1,)), ((), ())),
                          preferred_element_type=jnp.float32)
        + b1[...]
    )
    h = jnp.maximum(h, 0.0)
    out[...] = (
        lax.dot_general(h, w2[...], (((1,), (1,)), ((), ())),
                        preferred_element_type=jnp.float32)
        + b2[...]
    )


def kernel(node_features, edge_index, W1, b1, W2, b2):
    src = edge_index[0].astype(jnp.int32).reshape(NW, EPT)
    src = jnp.pad(src, ((0, 0), (0, EPT_P - EPT))).reshape(-1)
    tgt = edge_index[1].astype(jnp.int32).reshape(NW, EPT)
    tgt = jnp.pad(tgt, ((0, 0), (0, EPT_P - EPT)),
                  constant_values=N_PAD - 1).reshape(NW * NCH, 1, CH)
    zrows = jnp.zeros((CH, D), jnp.float32)
    ones = jnp.ones((CH, D), jnp.float32)

    psum, pcnt = _sc_aggregate(src, tgt, node_features, zrows, ones)

    R = 1000
    grid = (N_NODES // R,)
    out = pl.pallas_call(
        _tc_mlp_body,
        grid=grid,
        in_specs=[
            pl.BlockSpec((R, D), lambda i: (i, 0)),
            pl.BlockSpec((R, D), lambda i: (i, 0)),
            pl.BlockSpec((R, D), lambda i: (i, 0)),
            pl.BlockSpec((R, D), lambda i: (i, 0)),
            pl.BlockSpec((R, D), lambda i: (i, 0)),
            pl.BlockSpec((HID, D), lambda i: (0, 0)),
            pl.BlockSpec((HID, D), lambda i: (0, 0)),
            pl.BlockSpec((1, HID), lambda i: (0, 0)),
            pl.BlockSpec((D, HID), lambda i: (0, 0)),
            pl.BlockSpec((1, D), lambda i: (0, 0)),
        ],
        out_specs=pl.BlockSpec((R, D), lambda i: (i, 0)),
        out_shape=jax.ShapeDtypeStruct((N_NODES, D), jnp.float32),
    )(
        psum[:N_NODES], psum[N_PAD:N_PAD + N_NODES],
        pcnt[:N_NODES], pcnt[N_PAD:N_PAD + N_NODES],
        node_features,
        W1[:, :D], W1[:, D:],
        b1.reshape(1, HID), W2, b2.reshape(1, D),
    )
    return out
